# baseline (speedup 1.0000x reference)
import functools

import jax
import jax.numpy as jnp
from jax import lax
from jax.experimental import pallas as pl
from jax.experimental.pallas import tpu as pltpu
from jax.experimental.pallas import tpu_sc as plsc

_NC = 2
_NS = 16
_NW = _NC * _NS
_LANES = 16

_SC_BATCHES = 3


@functools.lru_cache(maxsize=None)
def _make_sc_add(nb: int, batch: int, seq: int, d: int, pos_rows: int):
    assert seq % _NW == 0
    rows = seq // _NW
    nvec = rows * d // _LANES
    npl = d // _LANES
    mesh = plsc.VectorSubcoreMesh(core_axis_name="c", subcore_axis_name="s")

    def body(x_hbm, pos_hbm, out_hbm, pos_v, buf0, buf1,
             in_sem0, in_sem1, out_sem0, out_sem1):
        wid = lax.axis_index("s") * _NC + lax.axis_index("c")
        base = wid * rows
        bufs = (buf0, buf1)
        in_sems = (in_sem0, in_sem1)
        out_sems = (out_sem0, out_sem1)
        in_cp = [None, None]
        out_cp = [None, None]

        def start_in(b, k):
            in_cp[k] = pltpu.async_copy(
                x_hbm.at[b, pl.ds(base, rows)], bufs[k], in_sems[k])

        start_in(0, 0)
        pltpu.sync_copy(pos_hbm.at[pl.ds(base, rows)], pos_v)

        for b in range(nb):
            k = b & 1
            if b + 1 < nb:
                if b >= 1:
                    out_cp[1 - k].wait()
                start_in(b + 1, 1 - k)
            in_cp[k].wait()
            buf = bufs[k]

            @plsc.parallel_loop(0, nvec, 1, unroll=8)
            def _(i):
                r = i // npl
                sl = pl.ds((i % npl) * _LANES, _LANES)
                plsc.addupdate(buf.at[r, sl], pos_v[r, sl])

            out_cp[k] = pltpu.async_copy(
                buf, out_hbm.at[b, pl.ds(base, rows)], out_sems[k])

        out_cp[0].wait()
        if nb > 1:
            out_cp[1].wait()

    return pl.kernel(
        body,
        out_type=jax.ShapeDtypeStruct((nb, seq, d), jnp.float32),
        mesh=mesh,
        scratch_types=[
            pltpu.VMEM((rows, d), jnp.float32),
            pltpu.VMEM((rows, d), jnp.float32),
            pltpu.VMEM((rows, d), jnp.float32),
            pltpu.SemaphoreType.DMA,
            pltpu.SemaphoreType.DMA,
            pltpu.SemaphoreType.DMA,
            pltpu.SemaphoreType.DMA,
        ],
    )


def _tc_body(x_ref, pos_ref, out_ref):
    out_ref[...] = x_ref[...] + pos_ref[...][None]


@functools.lru_cache(maxsize=None)
def _make_tc_add(b0: int, batch: int, seq: int, d: int, pos_rows: int):
    nb = batch - b0
    sblk = 256
    nsb = seq // sblk
    return pl.pallas_call(
        _tc_body,
        grid=(nsb, nb),
        in_specs=[
            pl.BlockSpec((1, sblk, d), lambda j, i: (b0 + i, j, 0)),
            pl.BlockSpec((sblk, d), lambda j, i: (j, 0)),
        ],
        out_specs=pl.BlockSpec((1, sblk, d), lambda j, i: (i, j, 0)),
        out_shape=jax.ShapeDtypeStruct((nb, seq, d), jnp.float32),
        compiler_params=pltpu.CompilerParams(
            dimension_semantics=("arbitrary", "arbitrary")),
    )


@jax.jit
def kernel(x, pos_embedding):
    batch, g, h, w, d = x.shape
    seq = g * h * w
    x3 = x.reshape(batch, seq, d)
    nsc = min(_SC_BATCHES, batch)
    out_sc = _make_sc_add(nsc, batch, seq, d, pos_embedding.shape[0])(
        x3, pos_embedding)
    if nsc < batch:
        out_tc = _make_tc_add(nsc, batch, seq, d, pos_embedding.shape[0])(
            x3, pos_embedding)
        out = jnp.concatenate([out_sc, out_tc], axis=0)
    else:
        out = out_sc
    return out.reshape(x.shape)


# speedup vs baseline: 4.7826x; 4.7826x over previous
import functools

import jax
import jax.numpy as jnp
from jax.experimental import pallas as pl


def _body(x_ref, pos_ref, out_ref):
    out_ref[...] = x_ref[...] + pos_ref[...][None]


@functools.lru_cache(maxsize=None)
def _make_add(batch: int, seq: int, d: int, pos_rows: int):
    bblk = 4 if batch % 4 == 0 else (2 if batch % 2 == 0 else 1)
    return pl.pallas_call(
        _body,
        grid=(batch // bblk,),
        in_specs=[
            pl.BlockSpec((bblk, seq, d), lambda i: (i, 0, 0)),
            pl.BlockSpec((seq, d), lambda i: (0, 0)),
        ],
        out_specs=pl.BlockSpec((bblk, seq, d), lambda i: (i, 0, 0)),
        out_shape=jax.ShapeDtypeStruct((batch, seq, d), jnp.float32),
    )


@jax.jit
def kernel(x, pos_embedding):
    batch, g, h, w, d = x.shape
    seq = g * h * w
    x3 = x.reshape(batch, seq, d)
    out = _make_add(batch, seq, d, pos_embedding.shape[0])(x3, pos_embedding)
    return out.reshape(x.shape)
